# baseline (speedup 1.0000x reference)
import functools

import jax
import jax.numpy as jnp
from jax import lax
from jax.experimental import pallas as pl
from jax.experimental.pallas import tpu as pltpu
from jax.experimental.pallas import tpu_sc as plsc

B = 16384
EMB = 64
LANES = 16

_info = plsc.get_sparse_core_info()
NC = _info.num_cores
NS = _info.num_subcores
NW = NC * NS
BPW = B // NW
CHUNK = 128
NCHUNK = BPW // CHUNK


@functools.partial(
    pl.kernel,
    out_type=jax.ShapeDtypeStruct((B,), jnp.float32),
    mesh=plsc.VectorSubcoreMesh(core_axis_name="c", subcore_axis_name="s"),
    compiler_params=pltpu.CompilerParams(use_tc_tiling_on_sc=False),
    scratch_types=[
        pltpu.VMEM((NCHUNK, CHUNK), jnp.int32),
        pltpu.VMEM((NCHUNK, CHUNK), jnp.int32),
        pltpu.VMEM((BPW, EMB), jnp.float32),
        pltpu.VMEM((BPW, EMB), jnp.float32),
        pltpu.VMEM((BPW,), jnp.float32),
        pltpu.SemaphoreType.DMA,
    ],
)
def _dot_kernel(uid_hbm, mid_hbm, ut_hbm, mt_hbm, out_hbm,
                uidx_v, midx_v, urows_v, mrows_v, out_v, sem):
    wid = lax.axis_index("s") * NC + lax.axis_index("c")

    pltpu.sync_copy(uid_hbm.at[wid], uidx_v)
    pltpu.sync_copy(mid_hbm.at[wid], midx_v)

    copies = []
    for g in range(NCHUNK):
        copies.append(pltpu.async_copy(
            ut_hbm.at[uidx_v.at[g]],
            urows_v.at[pl.ds(g * CHUNK, CHUNK)], sem))
        copies.append(pltpu.async_copy(
            mt_hbm.at[midx_v.at[g]],
            mrows_v.at[pl.ds(g * CHUNK, CHUNK)], sem))
    for c in copies:
        c.wait()

    lane = lax.iota(jnp.int32, 16)
    perms = [(lane + s) & (LANES - 1) for s in (8, 4, 2, 1)]
    gd = lax.GatherDimensionNumbers(
        offset_dims=(), collapsed_slice_dims=(0,), start_index_map=(0,))

    def shuffle(x, p):
        return lax.gather(x, p[:, None], gd, slice_sizes=(1,),
                          mode=lax.GatherScatterMode.PROMISE_IN_BOUNDS)

    def group_body(g, carry):
        base_r = g * LANES
        acc = jnp.zeros((LANES,), jnp.float32)
        for i in range(LANES):
            r = base_r + i
            t = urows_v[r, pl.ds(0, 16)] * mrows_v[r, pl.ds(0, 16)]
            for k in range(1, EMB // 16):
                t = t + (urows_v[r, pl.ds(k * 16, 16)]
                         * mrows_v[r, pl.ds(k * 16, 16)])
            for p in perms:
                t = t + shuffle(t, p)
            acc = jnp.where(lane == i, t, acc)
        out_v[pl.ds(base_r, LANES)] = acc
        return carry

    lax.fori_loop(0, BPW // LANES, group_body, 0)

    pltpu.sync_copy(out_v, out_hbm.at[pl.ds(wid * BPW, BPW)])


def kernel(userId, movieId, user_table, movie_table):
    uid = userId.reshape(NW, NCHUNK, CHUNK)
    mid = movieId.reshape(NW, NCHUNK, CHUNK)
    out = _dot_kernel(uid, mid, user_table, movie_table)
    return out.reshape(B, 1)
